# Initial kernel scaffold; baseline (speedup 1.0000x reference)
import jax
import jax.numpy as jnp
from jax.experimental import pallas as pl


def kernel(hv, segment_ids, W_gate, b_gate, W_proj, b_proj):
    raise NotImplementedError("write your pallas kernel here")


# speedup vs baseline: 5.1219x; 5.1219x over previous
import jax
import jax.numpy as jnp
from jax.experimental import pallas as pl
from jax.experimental.pallas import tpu as pltpu

N = 50000
D = 256
G = 1024
GH = 2 * D
BLK = 400
NBLK = N // BLK


def _fused_body(seg_ref, hv_ref, wg_ref, bg_ref, wp_ref, bp_ref, out_ref, acc_ref):
    i = pl.program_id(0)

    @pl.when(i == 0)
    def _init():
        acc_ref[...] = jnp.zeros_like(acc_ref)

    hv = hv_ref[...]
    wg = wg_ref[...]
    logits = jnp.sum(hv * wg, axis=1, keepdims=True) + bg_ref[0, 0]
    gate = 1.0 / (1.0 + jnp.exp(-logits))
    w = gate * hv
    gate_b = jnp.broadcast_to(gate, (BLK, 128))
    w_aug = jnp.concatenate([w, gate_b], axis=1).astype(jnp.bfloat16)

    seg = seg_ref[0]
    onehot = (seg == jax.lax.broadcasted_iota(jnp.int32, (BLK, G), 1)
              ).astype(jnp.bfloat16)
    acc_ref[...] += jax.lax.dot_general(
        onehot, w_aug, (((0,), (0,)), ((), ())),
        preferred_element_type=jnp.float32)

    @pl.when(i == NBLK - 1)
    def _final():
        s = acc_ref[:, :D]
        c = acc_ref[:, D:D + 1]
        out_ref[...] = jax.lax.dot_general(
            s, wp_ref[...], (((1,), (1,)), ((), ())),
            preferred_element_type=jnp.float32) + c * bp_ref[...]


def kernel(hv, segment_ids, W_gate, b_gate, W_proj, b_proj):
    seg3 = segment_ids.astype(jnp.int32).reshape(NBLK, BLK, 1)
    bg = b_gate.reshape(1, 1)
    bp = b_proj.reshape(1, GH)
    out = pl.pallas_call(
        _fused_body,
        grid=(NBLK,),
        in_specs=[
            pl.BlockSpec((1, BLK, 1), lambda i: (i, 0, 0)),
            pl.BlockSpec((BLK, D), lambda i: (i, 0)),
            pl.BlockSpec((1, D), lambda i: (0, 0)),
            pl.BlockSpec((1, 1), lambda i: (0, 0)),
            pl.BlockSpec((GH, D), lambda i: (0, 0)),
            pl.BlockSpec((1, GH), lambda i: (0, 0)),
        ],
        out_specs=pl.BlockSpec((G, GH), lambda i: (0, 0)),
        out_shape=jax.ShapeDtypeStruct((G, GH), jnp.float32),
        scratch_shapes=[pltpu.VMEM((G, D + 128), jnp.float32)],
    )(seg3, hv, W_gate, bg, W_proj, bp)
    return out
